# Initial kernel scaffold; baseline (speedup 1.0000x reference)
import jax
import jax.numpy as jnp
from jax.experimental import pallas as pl


def kernel(features, edge_index, v1, v2, W1, b1, W2, b2, W3, b3):
    raise NotImplementedError("write your pallas kernel here")


# speedup vs baseline: 2.2934x; 2.2934x over previous
import functools

import jax
import jax.numpy as jnp
from jax import lax
from jax.experimental import pallas as pl
from jax.experimental.pallas import tpu as pltpu
from jax.experimental.pallas import tpu_sc as plsc

N = 10000
E = 160000
D = 364
OUT = 150
B = 4096

N_PAD = 10240
D_PAD = 384
HALF = 192
QUAR = 96
E_PAD = 163840
EPT = E_PAD // 16
ECHUNK = 128
RPT = N_PAD // 16
OUT_PAD = 256
BB = 2 * B
VPT = BB // 16

_SC_MESH = dict(core_axis_name="c", subcore_axis_name="s")


def _leaky(x):
    return jnp.where(x > 0, x, 0.01 * x)


def _sc_segsum_body(xq, src_hbm, dst_hbm, out_hbm,
                    src_all, dst_all, gidx, didx, gbuf, acc, sem):
    c = lax.axis_index("c")
    s = lax.axis_index("s")
    iota = lax.iota(jnp.int32, 16)

    ebase = s * EPT
    pltpu.sync_copy(src_hbm.at[pl.ds(ebase, EPT)], src_all)
    pltpu.sync_copy(dst_hbm.at[pl.ds(ebase, EPT)], dst_all)
    rbase = s * RPT
    nchunks = EPT // ECHUNK

    for phase in range(2):
        q = 2 * phase + c

        for i in range(RPT // ECHUNK):
            b = i % 2
            for j in range(8):
                rows = iota + (rbase + i * ECHUNK + j * 16)
                gidx[b, pl.ds(j * 16, 16)] = rows * 4 + q
            pltpu.async_copy(xq.at[gidx.at[b]], gbuf.at[b], sem.at[b]).wait()
            pltpu.sync_copy(gbuf.at[b],
                            acc.at[pl.ds(rbase + i * ECHUNK, ECHUNK)])
        plsc.subcore_barrier()

        def fill(i, b):
            for j in range(8):
                off = i * ECHUNK + j * 16
                gidx[b, pl.ds(j * 16, 16)] = src_all[pl.ds(off, 16)] * 4 + q
                didx[b, pl.ds(j * 16, 16)] = dst_all[pl.ds(off, 16)]

        def gather(b):
            return pltpu.async_copy(xq.at[gidx.at[b]], gbuf.at[b], sem.at[b])

        def gather_wait(b):
            pltpu.make_async_copy(
                xq.at[gidx.at[b]], gbuf.at[b], sem.at[b]).wait()

        def scatter_add(b):
            pltpu.sync_copy(gbuf.at[b], acc.at[didx.at[b]], add=True)

        fill(0, 0)
        gather(0)

        def pair(k, _):
            i0 = 2 * k
            fill(i0 + 1, 1)
            gather_wait(0)
            gather(1)
            scatter_add(0)

            @pl.when(k < nchunks // 2 - 1)
            def _():
                fill(i0 + 2, 0)
                gather(0)
            gather_wait(1)
            scatter_add(1)
            return 0

        lax.fori_loop(0, nchunks // 2, pair, 0)
        plsc.subcore_barrier()

        for i in range(RPT // ECHUNK):
            b = i % 2
            pltpu.sync_copy(acc.at[pl.ds(rbase + i * ECHUNK, ECHUNK)],
                            gbuf.at[b])
            for j in range(8):
                rows = iota + (rbase + i * ECHUNK + j * 16)
                gidx[b, pl.ds(j * 16, 16)] = rows * 4 + q
            pltpu.async_copy(gbuf.at[b], out_hbm.at[gidx.at[b]],
                             sem.at[b]).wait()


@functools.cache
def _sc_segsum():
    return pl.kernel(
        _sc_segsum_body,
        mesh=plsc.VectorSubcoreMesh(**_SC_MESH),
        compiler_params=pltpu.CompilerParams(use_tc_tiling_on_sc=False),
        out_type=jax.ShapeDtypeStruct((4 * N_PAD, QUAR), jnp.float32),
        scratch_types=[
            pltpu.VMEM((EPT,), jnp.int32),
            pltpu.VMEM((EPT,), jnp.int32),
            pltpu.VMEM((2, ECHUNK), jnp.int32),
            pltpu.VMEM((2, ECHUNK), jnp.int32),
            pltpu.VMEM((2, ECHUNK, QUAR), jnp.float32),
            pltpu.VMEM_SHARED((N_PAD, QUAR), jnp.float32),
            pltpu.SemaphoreType.DMA((2,)),
        ],
    )


def _sc_gather_body(s2r, vcat_hbm, out_hbm, vcat_v, gidx, oidx, gbuf, sem):
    c = lax.axis_index("c")
    s = lax.axis_index("s")
    iota = lax.iota(jnp.int32, 16)
    vbase = s * VPT
    pltpu.sync_copy(vcat_hbm.at[pl.ds(vbase, VPT)], vcat_v)
    for k in range(VPT // ECHUNK):
        for j in range(8):
            off = k * ECHUNK + j * 16
            vv = vcat_v[pl.ds(off, 16)]
            gidx[pl.ds(j * 16, 16)] = vv * 2 + c
            oidx[pl.ds(j * 16, 16)] = (iota + vbase + off) * 2 + c
        pltpu.async_copy(s2r.at[gidx], gbuf, sem).wait()
        pltpu.async_copy(gbuf, out_hbm.at[oidx], sem).wait()


@functools.cache
def _sc_gather():
    return pl.kernel(
        _sc_gather_body,
        mesh=plsc.VectorSubcoreMesh(**_SC_MESH),
        compiler_params=pltpu.CompilerParams(use_tc_tiling_on_sc=False),
        out_type=jax.ShapeDtypeStruct((2 * BB, HALF), jnp.float32),
        scratch_types=[
            pltpu.VMEM((VPT,), jnp.int32),
            pltpu.VMEM((ECHUNK,), jnp.int32),
            pltpu.VMEM((ECHUNK,), jnp.int32),
            pltpu.VMEM((ECHUNK, HALF), jnp.float32),
            pltpu.SemaphoreType.DMA,
        ],
    )


def _tc_linear_body(s_ref, w_ref, b_ref, o_ref):
    acc = jnp.dot(s_ref[...], w_ref[...], preferred_element_type=jnp.float32)
    o_ref[...] = _leaky(acc + b_ref[...])


def _tc_linear(sarr, wt, b):
    n = sarr.shape[0]
    blk = 512
    return pl.pallas_call(
        _tc_linear_body,
        grid=(n // blk,),
        in_specs=[
            pl.BlockSpec((blk, D_PAD), lambda i: (i, 0)),
            pl.BlockSpec((D_PAD, D_PAD), lambda i: (0, 0)),
            pl.BlockSpec((1, D_PAD), lambda i: (0, 0)),
        ],
        out_specs=pl.BlockSpec((blk, D_PAD), lambda i: (i, 0)),
        out_shape=jax.ShapeDtypeStruct((n, D_PAD), jnp.float32),
    )(sarr, wt, b)


def _tc_final_body(u_ref, w2_ref, b2_ref, w3_ref, b3_ref, o_ref):
    t = jnp.dot(u_ref[...], w2_ref[...], preferred_element_type=jnp.float32)
    t = _leaky(t + b2_ref[...])
    z = jnp.dot(t, w3_ref[...], preferred_element_type=jnp.float32)
    z = _leaky(z + b3_ref[...])
    n = jnp.sqrt(jnp.sum(z * z, axis=1, keepdims=True))
    o_ref[...] = z / jnp.maximum(n, 1e-12)


def _tc_final(u, w2t, b2, w3t, b3):
    blk = 512
    return pl.pallas_call(
        _tc_final_body,
        grid=(BB // blk,),
        in_specs=[
            pl.BlockSpec((blk, D_PAD), lambda i: (i, 0)),
            pl.BlockSpec((D_PAD, D_PAD), lambda i: (0, 0)),
            pl.BlockSpec((1, D_PAD), lambda i: (0, 0)),
            pl.BlockSpec((D_PAD, OUT_PAD), lambda i: (0, 0)),
            pl.BlockSpec((1, OUT_PAD), lambda i: (0, 0)),
        ],
        out_specs=pl.BlockSpec((blk, OUT_PAD), lambda i: (i, 0)),
        out_shape=jax.ShapeDtypeStruct((BB, OUT_PAD), jnp.float32),
    )(u, w2t, b2, w3t, b3)


def kernel(features, edge_index, v1, v2, W1, b1, W2, b2, W3, b3):
    xp = jnp.pad(features, ((0, N_PAD - N), (0, D_PAD - D)))
    src = jnp.pad(edge_index[0], (0, E_PAD - E))
    dst = jnp.pad(edge_index[1], (0, E_PAD - E), constant_values=N_PAD - 1)
    vcat = jnp.concatenate([v1, v2])

    w1t = jnp.pad(W1, ((0, D_PAD - D), (0, D_PAD - D))).T
    b1p = jnp.pad(b1, (0, D_PAD - D)).reshape(1, D_PAD)
    w2t = jnp.pad(W2, ((0, D_PAD - D), (0, D_PAD - D))).T
    b2p = jnp.pad(b2, (0, D_PAD - D)).reshape(1, D_PAD)
    w3t = jnp.pad(W3, ((0, OUT_PAD - OUT), (0, D_PAD - D))).T
    b3p = jnp.pad(b3, (0, OUT_PAD - OUT)).reshape(1, OUT_PAD)

    s1q = _sc_segsum()(xp.reshape(4 * N_PAD, QUAR), src, dst)
    h = _tc_linear(s1q.reshape(N_PAD, D_PAD), w1t, b1p)

    s2q = _sc_segsum()(h.reshape(4 * N_PAD, QUAR), src, dst)

    ur = _sc_gather()(s2q.reshape(2 * N_PAD, HALF), vcat)
    z = _tc_final(ur.reshape(BB, D_PAD), w2t, b2p, w3t, b3p)
    return (z[:B, :OUT], z[B:, :OUT])
